# baseline (speedup 1.0000x reference)
import functools

import jax
import jax.numpy as jnp
import numpy as np
from jax import lax
from jax.experimental import pallas as pl
from jax.experimental.pallas import tpu as pltpu
from jax.experimental.pallas import tpu_sc as plsc


def _make_kernel(B, L, V, E):
    info = plsc.get_sparse_core_info()
    NC, NS, LANES = info.num_cores, info.num_subcores, info.num_lanes
    NW = NC * NS
    TB = 256
    B1 = TB // 128
    EV = E // LANES
    E8 = E // 8
    assert (B * L) % (NW * TB) == 0 and B % 128 == 0
    ntasks = (B * L) // (NW * TB)
    qmax = B // TB
    OBLK = E8 * B1 * 8 * 128
    gchunks = []
    off = 0
    while off < TB:
        sz = min(128, TB - off)
        gchunks.append((off, sz))
        off += sz

    mesh = plsc.VectorSubcoreMesh(core_axis_name="c", subcore_axis_name="s")

    @functools.partial(
        pl.kernel,
        mesh=mesh,
        out_type=jax.ShapeDtypeStruct((L, E8, B // 128, 8, 128), jnp.float32),
        scratch_types=[
            pltpu.VMEM((L, E), jnp.float32),
            [pltpu.VMEM((TB,), jnp.int32)] * 2,
            [pltpu.VMEM((TB, E), jnp.float32)] * 2,
            [pltpu.VMEM((E8, B1, 12, 129), jnp.float32)] * 2,
            [pltpu.SemaphoreType.DMA] * 6,
        ],
        compiler_params=pltpu.CompilerParams(
            use_tc_tiling_on_sc=False, needs_layout_passes=False),
    )
    def k(tok_hbm, xt_hbm, pos_hbm, out_hbm, pos_v, idx_v, rows_v, out_v, sems):
        wid = lax.axis_index("s") * NC + lax.axis_index("c")
        gsem, isem, wsem = sems[0:2], sems[2:4], sems[4:6]
        pltpu.sync_copy(pos_hbm, pos_v)
        t0 = wid * ntasks

        lane = lax.iota(jnp.int32, LANES)
        zero = lane >> 4
        i_e8 = [(lane >> 3) + v * (LANES // 8) for v in range(EV)]
        i_ein = lane & 7

        def locate(t):
            g = t0 + t
            return g // qmax, g % qmax

        def stage_idx(t, b):
            l, q = locate(t)
            pltpu.make_async_copy(
                xt_hbm.at[pl.ds(l * B + q * TB, TB)], idx_v[b], isem[b]).start()

        def iwait(b):
            pltpu.make_async_copy(
                xt_hbm.at[pl.ds(0, TB)], idx_v[b], isem[b]).wait()

        def start_gathers(b):
            for goff, gsz in gchunks:
                pltpu.make_async_copy(
                    tok_hbm.at[idx_v[b].at[pl.ds(goff, gsz)]],
                    rows_v[b].at[pl.ds(goff, gsz)], gsem[b]).start()

        def gwait(b):
            pltpu.make_async_copy(
                tok_hbm.at[pl.ds(0, TB)], rows_v[b], gsem[b]).wait()

        def start_wb(t, b):
            l, q = locate(t)
            for e8 in range(E8):
                for b1 in range(B1):
                    pltpu.make_async_copy(
                        out_v[b].at[e8, b1, pl.ds(0, 8), pl.ds(0, 128)],
                        out_hbm.at[l, e8, q * B1 + b1],
                        wsem[b]).start()

        def owait(b):
            pltpu.make_async_copy(
                out_v[b].at[:, :, pl.ds(0, 8), pl.ds(0, 128)],
                out_hbm.at[0, :, pl.ds(0, B1)], wsem[b]).wait()

        def compute(t, b):
            l, _ = locate(t)
            rows = rows_v[b]
            out = out_v[b]
            pv = [pos_v[l, pl.ds(v * LANES, LANES)] for v in range(EV)]

            for b1 in range(B1):
                i_b1 = zero + b1

                def tok_body(b0, carry):
                    j = b1 * 128 + b0
                    i_b0 = zero + b0
                    for v in range(EV):
                        y = rows[j, pl.ds(v * LANES, LANES)] + pv[v]
                        plsc.store_scatter(out, [i_e8[v], i_b1, i_ein, i_b0], y)
                    return carry

                lax.fori_loop(0, 128, tok_body, 0, unroll=8)

        def step(t, b, *, owait_b, gather_next, stage_next):
            nb = 1 - b
            gwait(b)
            if gather_next:
                iwait(nb)
                start_gathers(nb)
            if stage_next:
                stage_idx(t + 2, b)
            if owait_b:
                owait(b)
            compute(t, b)
            start_wb(t, b)

        stage_idx(0, 0)
        stage_idx(1, 1)
        iwait(0)
        start_gathers(0)
        step(0, 0, owait_b=False, gather_next=True, stage_next=True)
        step(1, 1, owait_b=False, gather_next=True, stage_next=True)

        def loop_body(i, carry):
            t = 2 + 2 * i
            step(t, 0, owait_b=True, gather_next=True, stage_next=True)
            step(t + 1, 1, owait_b=True, gather_next=True, stage_next=True)
            return carry

        lax.fori_loop(0, (ntasks - 4) // 2, loop_body, 0)
        step(ntasks - 2, 0, owait_b=True, gather_next=True, stage_next=False)
        step(ntasks - 1, 1, owait_b=True, gather_next=False, stage_next=False)
        owait(0)
        owait(1)

    return k


def kernel(x, tok_table, pos_table):
    B, L = x.shape
    V, E = tok_table.shape
    k = _make_kernel(B, L, V, E)
    out5 = k(tok_table, x.T.reshape(-1), pos_table)
    return out5.transpose(2, 4, 0, 1, 3).reshape(B, L, E)


# speedup vs baseline: 1.4588x; 1.4588x over previous
import functools

import jax
import jax.numpy as jnp
import numpy as np
from jax import lax
from jax.experimental import pallas as pl
from jax.experimental.pallas import tpu as pltpu
from jax.experimental.pallas import tpu_sc as plsc


def _make_kernel(B, L, V, E):
    info = plsc.get_sparse_core_info()
    NC, NS, LANES = info.num_cores, info.num_subcores, info.num_lanes
    NW = NC * NS
    TB = 256
    B1 = TB // 128
    EV = E // LANES
    E8 = E // 8
    assert (B * L) % (NW * TB) == 0 and B % 128 == 0
    ntasks = (B * L) // (NW * TB)
    qmax = B // TB
    OBLK = E8 * B1 * 8 * 128
    gchunks = []
    off = 0
    while off < TB:
        sz = min(128, TB - off)
        gchunks.append((off, sz))
        off += sz

    mesh = plsc.VectorSubcoreMesh(core_axis_name="c", subcore_axis_name="s")

    @functools.partial(
        pl.kernel,
        mesh=mesh,
        out_type=jax.ShapeDtypeStruct((L, E8, B // 128, 8, 128), jnp.float32),
        scratch_types=[
            pltpu.VMEM((L, E), jnp.float32),
            [pltpu.VMEM((TB,), jnp.int32)] * 2,
            [pltpu.VMEM((TB, E), jnp.float32)] * 2,
            [pltpu.VMEM((E8, B1, 12, 129), jnp.float32)] * 2,
            [pltpu.SemaphoreType.DMA] * 6,
        ],
        compiler_params=pltpu.CompilerParams(
            use_tc_tiling_on_sc=False, needs_layout_passes=False),
    )
    def k(tok_hbm, xt_hbm, pos_hbm, out_hbm, pos_v, idx_v, rows_v, out_v, sems):
        wid = lax.axis_index("s") * NC + lax.axis_index("c")
        gsem, isem, wsem = sems[0:2], sems[2:4], sems[4:6]
        pltpu.sync_copy(pos_hbm, pos_v)
        t0 = wid * ntasks

        lane = lax.iota(jnp.int32, LANES)
        zero = lane >> 4
        i_e8 = [(lane >> 3) + v * (LANES // 8) for v in range(EV)]
        i_ein = lane & 7

        def locate(t):
            g = t0 + t
            return g // qmax, g % qmax

        def stage_idx(t, b):
            l, q = locate(t)
            pltpu.make_async_copy(
                xt_hbm.at[pl.ds(l * B + q * TB, TB)], idx_v[b], isem[b]).start()

        def iwait(b):
            pltpu.make_async_copy(
                xt_hbm.at[pl.ds(0, TB)], idx_v[b], isem[b]).wait()

        def start_gathers(b):
            for goff, gsz in gchunks:
                pltpu.make_async_copy(
                    tok_hbm.at[idx_v[b].at[pl.ds(goff, gsz)]],
                    rows_v[b].at[pl.ds(goff, gsz)], gsem[b]).start()

        def gwait(b):
            pltpu.make_async_copy(
                tok_hbm.at[pl.ds(0, TB)], rows_v[b], gsem[b]).wait()

        def start_wb(t, b):
            l, q = locate(t)
            for e8 in range(E8):
                for b1 in range(B1):
                    pltpu.make_async_copy(
                        out_v[b].at[e8, b1, pl.ds(0, 8), pl.ds(0, 128)],
                        out_hbm.at[l, e8, q * B1 + b1],
                        wsem[b]).start()

        def owait(b):
            pltpu.make_async_copy(
                out_v[b].at[:, :, pl.ds(0, 8), pl.ds(0, 128)],
                out_hbm.at[0, :, pl.ds(0, B1)], wsem[b]).wait()

        def compute(t, b):
            l, _ = locate(t)
            rows = rows_v[b]
            out = out_v[b]
            pv = [pos_v[l, pl.ds(v * LANES, LANES)] for v in range(EV)]

            for b1 in range(B1):
                i_b1 = zero + b1

                @functools.partial(plsc.parallel_loop, 0, 128, unroll=8)
                def tok_body(b0):
                    j = b1 * 128 + b0
                    i_b0 = zero + b0
                    for v in range(EV):
                        y = rows[j, pl.ds(v * LANES, LANES)] + pv[v]
                        plsc.store_scatter(out, [i_e8[v], i_b1, i_ein, i_b0], y)

        def step(t, b, *, owait_b, gather_next, stage_next):
            nb = 1 - b
            gwait(b)
            if gather_next:
                iwait(nb)
                start_gathers(nb)
            if stage_next:
                stage_idx(t + 2, b)
            if owait_b:
                owait(b)
            compute(t, b)
            start_wb(t, b)

        stage_idx(0, 0)
        stage_idx(1, 1)
        iwait(0)
        start_gathers(0)
        step(0, 0, owait_b=False, gather_next=True, stage_next=True)
        step(1, 1, owait_b=False, gather_next=True, stage_next=True)

        def loop_body(i, carry):
            t = 2 + 2 * i
            step(t, 0, owait_b=True, gather_next=True, stage_next=True)
            step(t + 1, 1, owait_b=True, gather_next=True, stage_next=True)
            return carry

        lax.fori_loop(0, (ntasks - 4) // 2, loop_body, 0)
        step(ntasks - 2, 0, owait_b=True, gather_next=True, stage_next=False)
        step(ntasks - 1, 1, owait_b=True, gather_next=False, stage_next=False)
        owait(0)
        owait(1)

    return k


def kernel(x, tok_table, pos_table):
    B, L = x.shape
    V, E = tok_table.shape
    k = _make_kernel(B, L, V, E)
    out5 = k(tok_table, x.T.reshape(-1), pos_table)
    return out5.transpose(2, 4, 0, 1, 3).reshape(B, L, E)
